# baseline (speedup 1.0000x reference)
import numpy as np
import jax
import jax.numpy as jnp
from jax.experimental import pallas as pl

B, T, N, L = 4, 50, 30, 500
NPE = 16
NRBF = 16
NF = 32
EF = 32

_HIGHEST = jax.lax.Precision.HIGHEST


def _iota(shape, dim):
    return jax.lax.broadcasted_iota(jnp.int32, shape, dim).astype(jnp.float32)


def _nrm(x, eps=1e-12):
    n = jnp.sqrt(jnp.sum(x * x, axis=-1, keepdims=True))
    return x / jnp.maximum(n, eps)


def _cross(u, v):
    ux, uy, uz = u[..., 0:1], u[..., 1:2], u[..., 2:3]
    vx, vy, vz = v[..., 0:1], v[..., 1:2], v[..., 2:3]
    return jnp.concatenate(
        [uy * vz - uz * vy, uz * vx - ux * vz, ux * vy - uy * vx], axis=-1)


def _ln(x, g, b, eps=1e-5):
    mu = jnp.mean(x, axis=-1, keepdims=True)
    var = jnp.mean((x - mu) ** 2, axis=-1, keepdims=True)
    return g * (x - mu) / jnp.sqrt(var + eps) + b


def _dihed(u2, u1, u0):
    n2 = _nrm(_cross(u2, u1))
    n1 = _nrm(_cross(u1, u0))
    cosD = jnp.clip(jnp.sum(n2 * n1, axis=-1, keepdims=True),
                    -1.0 + 1e-7, 1.0 - 1e-7)
    sg_in = jnp.sum(u2 * n1, axis=-1, keepdims=True)
    sg = jnp.where(sg_in > 0.0, 1.0, jnp.where(sg_in < 0.0, -1.0, 0.0))
    return cosD, sg * jnp.sqrt(1.0 - cosD * cosD)


def _blockdiag3(w, k):
    z = jnp.zeros_like(w)
    rows = [jnp.concatenate([w if j == a else z for j in range(3)], axis=1)
            for a in range(3)]
    return jnp.concatenate(rows, axis=0)


NP = 32
NE = N * NP


def _dot(a, b, precision=_HIGHEST):
    return jax.lax.dot_general(a, b, (((1,), (0,)), ((), ())),
                               precision=precision)


def _tile_kernel(x_ref, xt_ref, focT_ref, ch_ref, const_ref,
                 whn_ref, wvn_ref, wsnw_ref, wsnb_ref,
                 whe_ref, wve_ref, wsew_ref, wseb_ref,
                 gn_ref, bn_ref, ge_ref, be_ref,
                 v_out_ref, e_out_ref, eidx_ref, aidx_ref):
    x = x_ref[0, 0]
    xt = xt_ref[0, 0]
    n_a = x[:, 0:3]
    ca = x[:, 3:6]
    c_a = x[:, 6:9]
    focT = focT_ref[0, 0]
    chain = ch_ref[0]
    E30 = const_ref[:, 0:N]
    r_flat = const_ref[:, N:N + 1]

    dx = ca[:, 0:1] - xt[3:4, :]
    dy = ca[:, 1:2] - xt[4:5, :]
    dz = ca[:, 2:3] - xt[5:6, :]
    D = jnp.sqrt(dx * dx + dy * dy + dz * dz + 1e-6)

    A = D[:, None, :]
    Bq = D[:, :, None]
    lt = (A < Bq).astype(jnp.float32)
    ktri = _iota((1, N, N), 2) < _iota((1, N, N), 1)
    eq = jnp.logical_and(A == Bq, ktri).astype(jnp.float32)
    rank = jnp.sum(lt + eq, axis=-1)

    rank_exp = _dot(E30, rank, precision=None)
    Pm = (rank_exp == r_flat).astype(jnp.float32)

    oh = (focT == _iota((N, L), 1)).astype(jnp.float32)
    nodechain = jnp.sum(oh * chain, axis=-1, keepdims=True)
    P0 = (rank == 0.0).astype(jnp.float32)
    first_chain = _dot(P0, nodechain)

    z1 = jnp.zeros((N, 1), jnp.float32)
    i_col = _iota((N, 1), 0)
    top = jnp.concatenate([ca, i_col, focT, focT, nodechain], axis=-1)
    bot = jnp.concatenate([-ca, z1, z1, -i_col, -first_chain], axis=-1)
    rhs = jnp.concatenate([top, bot], axis=0)
    lhs = jnp.concatenate([Pm, E30], axis=-1)
    G = _dot(lhs, rhs)

    eidx_col = jnp.floor(G[:, 3:4] + 0.5)
    aidx_col = jnp.floor(G[:, 4:5] + 0.5)
    d_rel = jnp.floor(G[:, 5:6] + 0.5)
    same = (jnp.abs(G[:, 6:7]) < 0.5).astype(jnp.float32)

    eidx_ref[0, 0] = eidx_col.reshape(N, NP, 1)[:, 0:N, :]
    aidx_ref[0, 0] = aidx_col.reshape(N, NP, 1)[:, 0:N, :]

    dnb = G[:, 0:3]
    ssq = jnp.sum(dnb * dnb, axis=-1, keepdims=True)
    Dn = jnp.sqrt(ssq + 1e-6)
    inv = 1.0 / jnp.maximum(jnp.sqrt(ssq), 1e-12)
    Ed = dnb * inv
    dir_ssq = jnp.sum(Ed * Ed, axis=-1, keepdims=True)

    sig = 20.0 / NRBF
    lane32 = _iota((1, 32), 1)
    f8 = lane32 - 16.0
    f8 = f8 - 8.0 * jnp.floor(f8 * 0.125)
    freq_row = jnp.where(lane32 < 16.0, 1.0 / sig,
                         jnp.exp(f8 * (-2.0 * np.log(10000.0) / NPE)))
    mu_shift = jnp.where(
        lane32 < 16.0, lane32 * (20.0 / (NRBF - 1)) / sig,
        jnp.where(lane32 < 24.0, 0.0, np.float32(np.pi / 2.0)))
    arg32 = _dot(jnp.concatenate([Dn, d_rel], axis=-1),
                 jnp.concatenate([jnp.where(lane32 < 16.0, freq_row, 0.0),
                                  jnp.where(lane32 < 16.0, 0.0, freq_row)],
                                 axis=0)) - mu_shift
    rbf_arg = arg32[:, 0:16]
    RBF = jnp.exp(-(rbf_arg * rbf_arg))
    Epos = jnp.cos(arg32[:, 16:32]) * same

    whe = whe_ref[...]
    w2 = _dot(whe, wve_ref[...])
    zr = jnp.zeros_like(w2)
    wexp = jnp.concatenate([
        jnp.concatenate([w2, zr, zr, zr], axis=-1),
        jnp.concatenate([zr, w2, zr, zr], axis=-1),
        jnp.concatenate([zr, zr, w2, zr], axis=-1),
        jnp.concatenate([zr, zr, zr, whe * whe], axis=-1)], axis=0)
    G2 = _dot(jnp.concatenate([Ed, dir_ssq], axis=-1), wexp)
    vo_e = G2[:, 0:96]
    vn_e = jnp.sqrt(G2[:, 96:128] + 1e-8)

    cat_e = jnp.concatenate([RBF, Epos, vn_e], axis=-1)
    so_e = _dot(cat_e, wsew_ref[...]) + wseb_ref[...]
    es = _ln(so_e, ge_ref[...], be_ref[...])
    e_full = jnp.concatenate([vo_e, es], axis=-1)
    e_out_ref[0, 0] = e_full.reshape(N, NP, 4 * EF)[:, 0:N, :]

    b0 = _nrm(ca - n_a)
    b1 = _nrm(c_a - ca)
    b2v = _nrm(n_a[1:] - c_a[:-1])
    zrow = jnp.zeros((1, 3), jnp.float32)
    U2 = jnp.concatenate([b2v, zrow], axis=0)
    U0n = jnp.concatenate([b0[1:], zrow], axis=0)
    U2p = jnp.concatenate([zrow, b2v], axis=0)
    c1, s1 = _dihed(b0, b1, U2)
    c2, s2 = _dihed(b1, U2, U0n)
    c0, s0 = _dihed(U2p, b0, b1)
    row = _iota((N, 1), 0)
    lo, hi = row >= 1.0, row <= (N - 2.0)
    cos3 = jnp.concatenate([jnp.where(lo, c0, 1.0), jnp.where(hi, c1, 1.0),
                            jnp.where(hi, c2, 1.0)], axis=-1)
    sin3 = jnp.concatenate([jnp.where(lo, s0, 0.0), jnp.where(hi, s1, 0.0),
                            jnp.where(hi, s2, 0.0)], axis=-1)
    V_dih = jnp.concatenate([cos3, sin3], axis=-1)

    fw_core = _nrm(ca[1:] - ca[:-1])
    fw = jnp.concatenate([fw_core, zrow], axis=0)
    bw = jnp.concatenate([zrow, -fw_core], axis=0)

    cdir = _nrm(c_a - ca)
    ndir = _nrm(n_a - ca)
    bis = _nrm(cdir + ndir)
    perp = _nrm(_cross(cdir, ndir))
    vec = -bis * np.sqrt(1.0 / 3.0) - perp * np.sqrt(2.0 / 3.0)

    vparts = []
    for a in range(3):
        vparts += [vec[:, a:a + 1], fw[:, a:a + 1], bw[:, a:a + 1]]
    v_n = jnp.concatenate(vparts, axis=-1)

    W9 = _blockdiag3(whn_ref[...], 3)
    vh96 = jax.lax.dot_general(v_n, W9, (((1,), (0,)), ((), ())),
                               precision=_HIGHEST)
    vn_n = jnp.sqrt(vh96[:, 0:NF] ** 2 + vh96[:, NF:2 * NF] ** 2 +
                    vh96[:, 2 * NF:3 * NF] ** 2 + 1e-8)
    so_n = jax.lax.dot_general(
        jnp.concatenate([V_dih, vn_n], axis=-1), wsnw_ref[...],
        (((1,), (0,)), ((), ())), precision=_HIGHEST) + wsnb_ref[...]
    W96 = _blockdiag3(wvn_ref[...], NF)
    vo96 = jax.lax.dot_general(vh96, W96, (((1,), (0,)), ((), ())),
                               precision=_HIGHEST)
    vs = _ln(so_n, gn_ref[...], bn_ref[...])
    v_out_ref[0, 0] = jnp.concatenate([vo96, vs], axis=-1)


def kernel(X, mask, Wh_n, Wv_n, Wsn_w, Wsn_b, Wh_e, Wv_e, Wse_w, Wse_b,
           g_n, b_n, g_e, b_e, chain_idx, batched_focuses):
    del mask
    Xr = X.reshape(B, T, N, 12)
    Xt = jnp.swapaxes(Xr, 2, 3)
    focT = batched_focuses.astype(jnp.float32).reshape(B, T, N, 1)
    ch = chain_idx.astype(jnp.float32).reshape(B, 1, L)
    q = np.arange(N * NP)
    const = jnp.asarray(np.concatenate(
        [(q[:, None] // NP == np.arange(N)[None, :]).astype(np.float32),
         (q[:, None] % NP).astype(np.float32),
         np.zeros((N * NP, 1), np.float32)], axis=1))

    def row(w):
        return w.reshape(1, -1)

    full = lambda shape: pl.BlockSpec(shape, lambda b, t: (0,) * len(shape))
    in_specs = [
        pl.BlockSpec((1, 1, N, 12), lambda b, t: (b, t, 0, 0)),
        pl.BlockSpec((1, 1, 12, N), lambda b, t: (b, t, 0, 0)),
        pl.BlockSpec((1, 1, N, 1), lambda b, t: (b, t, 0, 0)),
        pl.BlockSpec((1, 1, L), lambda b, t: (b, 0, 0)),
        full((N * NP, NP)),
        full((3, NF)),
        full((NF, NF)),
        full((6 + NF, NF)),
        full((1, NF)),
        full((1, NF)),
        full((NF, EF)),
        full((NRBF * 2 + NF, EF)),
        full((1, EF)),
        full((1, NF)),
        full((1, NF)),
        full((1, EF)),
        full((1, EF)),
    ]
    out_specs = [
        pl.BlockSpec((1, 1, N, 4 * NF), lambda b, t: (b, t, 0, 0)),
        pl.BlockSpec((1, 1, N, N, 4 * EF), lambda b, t: (b, t, 0, 0, 0)),
        pl.BlockSpec((1, 1, N, N, 1), lambda b, t: (b, t, 0, 0, 0)),
        pl.BlockSpec((1, 1, N, N, 1), lambda b, t: (b, t, 0, 0, 0)),
    ]
    out_shapes = [
        jax.ShapeDtypeStruct((B, T, N, 4 * NF), jnp.float32),
        jax.ShapeDtypeStruct((B, T, N, N, 4 * EF), jnp.float32),
        jax.ShapeDtypeStruct((B, T, N, N, 1), jnp.float32),
        jax.ShapeDtypeStruct((B, T, N, N, 1), jnp.float32),
    ]
    V, E, eidx_f, aidx_f = pl.pallas_call(
        _tile_kernel,
        grid=(B, T),
        in_specs=in_specs,
        out_specs=out_specs,
        out_shape=out_shapes,
    )(Xr, Xt, focT, ch, const, Wh_n, Wv_n, Wsn_w, row(Wsn_b), row(Wh_e.reshape(-1)),
      Wv_e, Wse_w, row(Wse_b), row(g_n), row(b_n), row(g_e), row(b_e))
    return (V, E,
            eidx_f.reshape(B, T, N, N).astype(jnp.int32),
            aidx_f.reshape(B, T, N, N).astype(jnp.int32))


# speedup vs baseline: 1.0645x; 1.0645x over previous
import numpy as np
import jax
import jax.numpy as jnp
from jax.experimental import pallas as pl

B, T, N, L = 4, 50, 30, 500
NPE = 16
NRBF = 16
NF = 32
EF = 32

_HIGHEST = jax.lax.Precision.HIGHEST


def _iota(shape, dim):
    return jax.lax.broadcasted_iota(jnp.int32, shape, dim).astype(jnp.float32)


def _nrm(x, eps=1e-12):
    n = jnp.sqrt(jnp.sum(x * x, axis=-1, keepdims=True))
    return x / jnp.maximum(n, eps)


def _cross(u, v):
    ux, uy, uz = u[..., 0:1], u[..., 1:2], u[..., 2:3]
    vx, vy, vz = v[..., 0:1], v[..., 1:2], v[..., 2:3]
    return jnp.concatenate(
        [uy * vz - uz * vy, uz * vx - ux * vz, ux * vy - uy * vx], axis=-1)


def _ln(x, g, b, eps=1e-5):
    mu = jnp.mean(x, axis=-1, keepdims=True)
    var = jnp.mean((x - mu) ** 2, axis=-1, keepdims=True)
    return g * (x - mu) / jnp.sqrt(var + eps) + b


def _dihed(u2, u1, u0):
    n2 = _nrm(_cross(u2, u1))
    n1 = _nrm(_cross(u1, u0))
    cosD = jnp.clip(jnp.sum(n2 * n1, axis=-1, keepdims=True),
                    -1.0 + 1e-7, 1.0 - 1e-7)
    sg_in = jnp.sum(u2 * n1, axis=-1, keepdims=True)
    sg = jnp.where(sg_in > 0.0, 1.0, jnp.where(sg_in < 0.0, -1.0, 0.0))
    return cosD, sg * jnp.sqrt(1.0 - cosD * cosD)


def _blockdiag3(w, k):
    z = jnp.zeros_like(w)
    rows = [jnp.concatenate([w if j == a else z for j in range(3)], axis=1)
            for a in range(3)]
    return jnp.concatenate(rows, axis=0)


NP = 32
NE = N * NP


def _dot(a, b, precision=_HIGHEST):
    return jax.lax.dot_general(a, b, (((1,), (0,)), ((), ())),
                               precision=precision)


def _tile_kernel(x_ref, xt_ref, focT_ref, ch_ref, const_ref,
                 whn_ref, wvn_ref, wsnw_ref, wsnb_ref,
                 whe_ref, wve_ref, wsew_ref, wseb_ref,
                 gn_ref, bn_ref, ge_ref, be_ref,
                 v_out_ref, e_out_ref, eidx_ref, aidx_ref):
    x = x_ref[0, 0]
    xt = xt_ref[0, 0]
    n_a = x[:, 0:3]
    ca = x[:, 3:6]
    c_a = x[:, 6:9]
    focT = focT_ref[0, 0]
    chain = ch_ref[0]
    E30 = const_ref[:, 0:N]
    r_flat = const_ref[:, N:N + 1]

    dx = ca[:, 0:1] - xt[3:4, :]
    dy = ca[:, 1:2] - xt[4:5, :]
    dz = ca[:, 2:3] - xt[5:6, :]
    D = jnp.sqrt(dx * dx + dy * dy + dz * dz + 1e-6)

    A = D[:, None, :]
    Bq = D[:, :, None]
    lt = (A < Bq).astype(jnp.float32)
    ktri = _iota((1, N, N), 2) < _iota((1, N, N), 1)
    eq = jnp.logical_and(A == Bq, ktri).astype(jnp.float32)
    rank = jnp.sum(lt + eq, axis=-1)

    rank_exp = _dot(E30, rank, precision=None)
    Pm = (rank_exp == r_flat).astype(jnp.float32)

    oh = (focT == _iota((N, L), 1)).astype(jnp.float32)
    nodechain = jnp.sum(oh * chain, axis=-1, keepdims=True)
    P0 = (rank == 0.0).astype(jnp.float32)
    first_chain = _dot(P0, nodechain)

    z1 = jnp.zeros((N, 1), jnp.float32)
    i_col = _iota((N, 1), 0)
    top = jnp.concatenate([ca, i_col, focT, focT, nodechain], axis=-1)
    bot = jnp.concatenate([-ca, z1, z1, -i_col, -first_chain], axis=-1)
    rhs = jnp.concatenate([top, bot], axis=0)
    lhs = jnp.concatenate([Pm, E30], axis=-1)
    G = _dot(lhs, rhs)

    flo = jnp.floor(G[:, 3:6] + 0.5)
    eidx_col = flo[:, 0:1]
    aidx_col = flo[:, 1:2]
    d_rel = flo[:, 2:3]
    same = (jnp.abs(G[:, 6:7]) < 0.5).astype(jnp.float32)
    same16 = _dot(same, jnp.ones((1, NRBF), jnp.float32), precision=None)

    eidx_ref[0, 0] = eidx_col.reshape(N, NP, 1)[:, 0:N, :]
    aidx_ref[0, 0] = aidx_col.reshape(N, NP, 1)[:, 0:N, :]

    dnb = G[:, 0:3]
    ssq = _dot(dnb * dnb, jnp.ones((3, 1), jnp.float32))
    sq2 = jnp.sqrt(jnp.concatenate([ssq + 1e-6, ssq], axis=-1))
    Dn = sq2[:, 0:1]
    inv = 1.0 / jnp.maximum(sq2[:, 1:2], 1e-12)
    Ed = dnb * inv
    dir_ssq = ssq * (inv * inv)

    sig = 20.0 / NRBF
    lane32 = _iota((1, 32), 1)
    f8 = lane32 - 16.0
    f8 = f8 - 8.0 * jnp.floor(f8 * 0.125)
    freq_row = jnp.where(lane32 < 16.0, 1.0 / sig,
                         jnp.exp(f8 * (-2.0 * np.log(10000.0) / NPE)))
    mu_shift = jnp.where(
        lane32 < 16.0, lane32 * (20.0 / (NRBF - 1)) / sig,
        jnp.where(lane32 < 24.0, 0.0, np.float32(np.pi / 2.0)))
    arg32 = _dot(jnp.concatenate([Dn, d_rel], axis=-1),
                 jnp.concatenate([jnp.where(lane32 < 16.0, freq_row, 0.0),
                                  jnp.where(lane32 < 16.0, 0.0, freq_row)],
                                 axis=0)) - mu_shift
    rbf_arg = arg32[:, 0:16]
    RBF = jnp.exp(-(rbf_arg * rbf_arg))
    Epos = jnp.cos(arg32[:, 16:32]) * same16

    whe = whe_ref[...]
    w2 = _dot(whe, wve_ref[...])
    zr = jnp.zeros_like(w2)
    wexp = jnp.concatenate([
        jnp.concatenate([w2, zr, zr, zr], axis=-1),
        jnp.concatenate([zr, w2, zr, zr], axis=-1),
        jnp.concatenate([zr, zr, w2, zr], axis=-1),
        jnp.concatenate([zr, zr, zr, whe * whe], axis=-1)], axis=0)
    G2 = _dot(jnp.concatenate([Ed, dir_ssq], axis=-1), wexp)
    vo_e = G2[:, 0:96]
    vn_e = jnp.sqrt(G2[:, 96:128] + 1e-8)

    cat_e = jnp.concatenate([RBF, Epos, vn_e], axis=-1)
    so_e = _dot(cat_e, wsew_ref[...]) + wseb_ref[...]
    es = _ln(so_e, ge_ref[...], be_ref[...])
    e_full = jnp.concatenate([vo_e, es], axis=-1)
    e_out_ref[0, 0] = e_full.reshape(N, NP, 4 * EF)[:, 0:N, :]

    b0 = _nrm(ca - n_a)
    b1 = _nrm(c_a - ca)
    b2v = _nrm(n_a[1:] - c_a[:-1])
    zrow = jnp.zeros((1, 3), jnp.float32)
    U2 = jnp.concatenate([b2v, zrow], axis=0)
    U0n = jnp.concatenate([b0[1:], zrow], axis=0)
    U2p = jnp.concatenate([zrow, b2v], axis=0)
    c1, s1 = _dihed(b0, b1, U2)
    c2, s2 = _dihed(b1, U2, U0n)
    c0, s0 = _dihed(U2p, b0, b1)
    row = _iota((N, 1), 0)
    lo, hi = row >= 1.0, row <= (N - 2.0)
    cos3 = jnp.concatenate([jnp.where(lo, c0, 1.0), jnp.where(hi, c1, 1.0),
                            jnp.where(hi, c2, 1.0)], axis=-1)
    sin3 = jnp.concatenate([jnp.where(lo, s0, 0.0), jnp.where(hi, s1, 0.0),
                            jnp.where(hi, s2, 0.0)], axis=-1)
    V_dih = jnp.concatenate([cos3, sin3], axis=-1)

    fw_core = _nrm(ca[1:] - ca[:-1])
    fw = jnp.concatenate([fw_core, zrow], axis=0)
    bw = jnp.concatenate([zrow, -fw_core], axis=0)

    cdir = _nrm(c_a - ca)
    ndir = _nrm(n_a - ca)
    bis = _nrm(cdir + ndir)
    perp = _nrm(_cross(cdir, ndir))
    vec = -bis * np.sqrt(1.0 / 3.0) - perp * np.sqrt(2.0 / 3.0)

    vparts = []
    for a in range(3):
        vparts += [vec[:, a:a + 1], fw[:, a:a + 1], bw[:, a:a + 1]]
    v_n = jnp.concatenate(vparts, axis=-1)

    W9 = _blockdiag3(whn_ref[...], 3)
    vh96 = jax.lax.dot_general(v_n, W9, (((1,), (0,)), ((), ())),
                               precision=_HIGHEST)
    vn_n = jnp.sqrt(vh96[:, 0:NF] ** 2 + vh96[:, NF:2 * NF] ** 2 +
                    vh96[:, 2 * NF:3 * NF] ** 2 + 1e-8)
    so_n = jax.lax.dot_general(
        jnp.concatenate([V_dih, vn_n], axis=-1), wsnw_ref[...],
        (((1,), (0,)), ((), ())), precision=_HIGHEST) + wsnb_ref[...]
    W96 = _blockdiag3(wvn_ref[...], NF)
    vo96 = jax.lax.dot_general(vh96, W96, (((1,), (0,)), ((), ())),
                               precision=_HIGHEST)
    vs = _ln(so_n, gn_ref[...], bn_ref[...])
    v_out_ref[0, 0] = jnp.concatenate([vo96, vs], axis=-1)


def kernel(X, mask, Wh_n, Wv_n, Wsn_w, Wsn_b, Wh_e, Wv_e, Wse_w, Wse_b,
           g_n, b_n, g_e, b_e, chain_idx, batched_focuses):
    del mask
    Xr = X.reshape(B, T, N, 12)
    Xt = jnp.swapaxes(Xr, 2, 3)
    focT = batched_focuses.astype(jnp.float32).reshape(B, T, N, 1)
    ch = chain_idx.astype(jnp.float32).reshape(B, 1, L)
    q = np.arange(N * NP)
    const = jnp.asarray(np.concatenate(
        [(q[:, None] // NP == np.arange(N)[None, :]).astype(np.float32),
         (q[:, None] % NP).astype(np.float32),
         np.zeros((N * NP, 1), np.float32)], axis=1))

    def row(w):
        return w.reshape(1, -1)

    full = lambda shape: pl.BlockSpec(shape, lambda b, t: (0,) * len(shape))
    in_specs = [
        pl.BlockSpec((1, 1, N, 12), lambda b, t: (b, t, 0, 0)),
        pl.BlockSpec((1, 1, 12, N), lambda b, t: (b, t, 0, 0)),
        pl.BlockSpec((1, 1, N, 1), lambda b, t: (b, t, 0, 0)),
        pl.BlockSpec((1, 1, L), lambda b, t: (b, 0, 0)),
        full((N * NP, NP)),
        full((3, NF)),
        full((NF, NF)),
        full((6 + NF, NF)),
        full((1, NF)),
        full((1, NF)),
        full((NF, EF)),
        full((NRBF * 2 + NF, EF)),
        full((1, EF)),
        full((1, NF)),
        full((1, NF)),
        full((1, EF)),
        full((1, EF)),
    ]
    out_specs = [
        pl.BlockSpec((1, 1, N, 4 * NF), lambda b, t: (b, t, 0, 0)),
        pl.BlockSpec((1, 1, N, N, 4 * EF), lambda b, t: (b, t, 0, 0, 0)),
        pl.BlockSpec((1, 1, N, N, 1), lambda b, t: (b, t, 0, 0, 0)),
        pl.BlockSpec((1, 1, N, N, 1), lambda b, t: (b, t, 0, 0, 0)),
    ]
    out_shapes = [
        jax.ShapeDtypeStruct((B, T, N, 4 * NF), jnp.float32),
        jax.ShapeDtypeStruct((B, T, N, N, 4 * EF), jnp.float32),
        jax.ShapeDtypeStruct((B, T, N, N, 1), jnp.float32),
        jax.ShapeDtypeStruct((B, T, N, N, 1), jnp.float32),
    ]
    V, E, eidx_f, aidx_f = pl.pallas_call(
        _tile_kernel,
        grid=(B, T),
        in_specs=in_specs,
        out_specs=out_specs,
        out_shape=out_shapes,
    )(Xr, Xt, focT, ch, const, Wh_n, Wv_n, Wsn_w, row(Wsn_b), row(Wh_e.reshape(-1)),
      Wv_e, Wse_w, row(Wse_b), row(g_n), row(b_n), row(g_e), row(b_e))
    return (V, E,
            eidx_f.reshape(B, T, N, N).astype(jnp.int32),
            aidx_f.reshape(B, T, N, N).astype(jnp.int32))


# speedup vs baseline: 1.2031x; 1.1302x over previous
import numpy as np
import jax
import jax.numpy as jnp
from jax.experimental import pallas as pl

B, T, N, L = 4, 50, 30, 500
NPE = 16
NRBF = 16
NF = 32
EF = 32

_HIGHEST = jax.lax.Precision.HIGHEST


def _iota(shape, dim):
    return jax.lax.broadcasted_iota(jnp.int32, shape, dim).astype(jnp.float32)


def _nrm(x, eps=1e-12):
    n = jnp.sqrt(jnp.sum(x * x, axis=-1, keepdims=True))
    return x / jnp.maximum(n, eps)


def _cross(u, v):
    ux, uy, uz = u[..., 0:1], u[..., 1:2], u[..., 2:3]
    vx, vy, vz = v[..., 0:1], v[..., 1:2], v[..., 2:3]
    return jnp.concatenate(
        [uy * vz - uz * vy, uz * vx - ux * vz, ux * vy - uy * vx], axis=-1)


def _ln(x, g, b, eps=1e-5):
    mu = jnp.mean(x, axis=-1, keepdims=True)
    var = jnp.mean((x - mu) ** 2, axis=-1, keepdims=True)
    return g * (x - mu) / jnp.sqrt(var + eps) + b


def _dihed(u2, u1, u0):
    n2 = _nrm(_cross(u2, u1))
    n1 = _nrm(_cross(u1, u0))
    cosD = jnp.clip(jnp.sum(n2 * n1, axis=-1, keepdims=True),
                    -1.0 + 1e-7, 1.0 - 1e-7)
    sg_in = jnp.sum(u2 * n1, axis=-1, keepdims=True)
    sg = jnp.where(sg_in > 0.0, 1.0, jnp.where(sg_in < 0.0, -1.0, 0.0))
    return cosD, sg * jnp.sqrt(1.0 - cosD * cosD)


def _blockdiag3(w, k):
    z = jnp.zeros_like(w)
    rows = [jnp.concatenate([w if j == a else z for j in range(3)], axis=1)
            for a in range(3)]
    return jnp.concatenate(rows, axis=0)


NP = 32
NE = N * NP


def _dot(a, b, precision=_HIGHEST):
    return jax.lax.dot_general(a, b, (((1,), (0,)), ((), ())),
                               precision=precision)


def _tile_kernel(x_ref, xt_ref, focT_ref, ch_ref, const_ref, trig_ref,
                 whn_ref, wvn_ref, wsnw_ref, wsnb_ref,
                 whe_ref, wve_ref, wsew_ref, wseb_ref,
                 gn_ref, bn_ref, ge_ref, be_ref,
                 v_out_ref, e_out_ref, eidx_ref, aidx_ref):
    x = x_ref[0, 0]
    xt = xt_ref[0, 0]
    n_a = x[:, 0:3]
    ca = x[:, 3:6]
    c_a = x[:, 6:9]
    focT = focT_ref[0, 0]
    chain = ch_ref[0]
    E30 = const_ref[:, 0:N]
    r_flat = const_ref[:, N:N + 1]

    dx = ca[:, 0:1] - xt[3:4, :]
    dy = ca[:, 1:2] - xt[4:5, :]
    dz = ca[:, 2:3] - xt[5:6, :]
    D = jnp.sqrt(dx * dx + dy * dy + dz * dz + 1e-6)

    A = D[:, None, :]
    Bq = D[:, :, None]
    lt = (A < Bq).astype(jnp.float32)
    ktri = _iota((1, N, N), 2) < _iota((1, N, N), 1)
    eq = jnp.logical_and(A == Bq, ktri).astype(jnp.float32)
    rank = jnp.sum(lt + eq, axis=-1)

    rank_exp = _dot(E30, rank, precision=None)
    Pm = (rank_exp == r_flat).astype(jnp.float32)

    oh = (focT == _iota((N, L), 1)).astype(jnp.float32)
    nodechain = jnp.sum(oh * chain, axis=-1, keepdims=True)
    P0 = (rank == 0.0).astype(jnp.float32)
    first_chain = _dot(P0, nodechain)

    freq8 = jnp.exp(_iota((1, 8), 1) * (-2.0 * np.log(10000.0) / NPE))
    fa = focT * freq8
    cjsj = jnp.cos(jnp.concatenate([fa, fa - np.float32(np.pi / 2)], axis=-1))
    cisi = trig_ref[...]

    z1 = jnp.zeros((N, 1), jnp.float32)
    z16 = jnp.zeros((N, 16), jnp.float32)
    i_col = _iota((N, 1), 0)
    top = jnp.concatenate([ca, i_col, focT, nodechain, cjsj, z16], axis=-1)
    bot = jnp.concatenate([-ca, z1, z1, -first_chain, z16, cisi], axis=-1)
    rhs = jnp.concatenate([top, bot], axis=0)
    lhs = jnp.concatenate([Pm, E30], axis=-1)
    G = _dot(lhs, rhs)

    flo = jnp.floor(G[:, 3:5] + 0.5)
    eidx_col = flo[:, 0:1]
    aidx_col = flo[:, 1:2]
    same = (jnp.abs(G[:, 5:6]) < 0.5).astype(jnp.float32)
    same16 = _dot(same, jnp.ones((1, NRBF), jnp.float32), precision=None)

    eidx_ref[0, 0] = eidx_col.reshape(N, NP, 1)[:, 0:N, :]
    aidx_ref[0, 0] = aidx_col.reshape(N, NP, 1)[:, 0:N, :]

    dnb = G[:, 0:3]
    ssq = _dot(dnb * dnb, jnp.ones((3, 1), jnp.float32))
    sq2 = jnp.sqrt(jnp.concatenate([ssq + 1e-6, ssq], axis=-1))
    Dn = sq2[:, 0:1]
    inv = 1.0 / jnp.maximum(sq2[:, 1:2], 1e-12)
    Ed = dnb * inv
    dir_ssq = ssq * (inv * inv)

    sig = 20.0 / NRBF
    lane16 = _iota((1, 16), 1)
    rbf_arg = _dot(Dn, jnp.full((1, 16), 1.0 / sig, jnp.float32)) \
        - lane16 * ((20.0 / (NRBF - 1)) / sig)
    RBF = jnp.exp(-(rbf_arg * rbf_arg))
    cjE, sjE = G[:, 6:14], G[:, 14:22]
    ciE, siE = G[:, 22:30], G[:, 30:38]
    Epos = jnp.concatenate([cjE * ciE + sjE * siE,
                            sjE * ciE - cjE * siE], axis=-1) * same16

    whe = whe_ref[...]
    w2 = _dot(whe, wve_ref[...])
    zr = jnp.zeros_like(w2)
    wexp = jnp.concatenate([
        jnp.concatenate([w2, zr, zr, zr], axis=-1),
        jnp.concatenate([zr, w2, zr, zr], axis=-1),
        jnp.concatenate([zr, zr, w2, zr], axis=-1),
        jnp.concatenate([zr, zr, zr, whe * whe], axis=-1)], axis=0)
    G2 = _dot(jnp.concatenate([Ed, dir_ssq], axis=-1), wexp)
    vo_e = G2[:, 0:96]
    vn_e = jnp.sqrt(G2[:, 96:128] + 1e-8)

    cat_e = jnp.concatenate([RBF, Epos, vn_e], axis=-1)
    so_e = _dot(cat_e, wsew_ref[...]) + wseb_ref[...]
    es = _ln(so_e, ge_ref[...], be_ref[...])
    e_full = jnp.concatenate([vo_e, es], axis=-1)
    e_out_ref[0, 0] = e_full.reshape(N, NP, 4 * EF)[:, 0:N, :]

    b0 = _nrm(ca - n_a)
    b1 = _nrm(c_a - ca)
    b2v = _nrm(n_a[1:] - c_a[:-1])
    zrow = jnp.zeros((1, 3), jnp.float32)
    U2 = jnp.concatenate([b2v, zrow], axis=0)
    U0n = jnp.concatenate([b0[1:], zrow], axis=0)
    U2p = jnp.concatenate([zrow, b2v], axis=0)
    c1, s1 = _dihed(b0, b1, U2)
    c2, s2 = _dihed(b1, U2, U0n)
    c0, s0 = _dihed(U2p, b0, b1)
    row = _iota((N, 1), 0)
    lo, hi = row >= 1.0, row <= (N - 2.0)
    cos3 = jnp.concatenate([jnp.where(lo, c0, 1.0), jnp.where(hi, c1, 1.0),
                            jnp.where(hi, c2, 1.0)], axis=-1)
    sin3 = jnp.concatenate([jnp.where(lo, s0, 0.0), jnp.where(hi, s1, 0.0),
                            jnp.where(hi, s2, 0.0)], axis=-1)
    V_dih = jnp.concatenate([cos3, sin3], axis=-1)

    fw_core = _nrm(ca[1:] - ca[:-1])
    fw = jnp.concatenate([fw_core, zrow], axis=0)
    bw = jnp.concatenate([zrow, -fw_core], axis=0)

    cdir = _nrm(c_a - ca)
    ndir = _nrm(n_a - ca)
    bis = _nrm(cdir + ndir)
    perp = _nrm(_cross(cdir, ndir))
    vec = -bis * np.sqrt(1.0 / 3.0) - perp * np.sqrt(2.0 / 3.0)

    vparts = []
    for a in range(3):
        vparts += [vec[:, a:a + 1], fw[:, a:a + 1], bw[:, a:a + 1]]
    v_n = jnp.concatenate(vparts, axis=-1)

    W9 = _blockdiag3(whn_ref[...], 3)
    vh96 = jax.lax.dot_general(v_n, W9, (((1,), (0,)), ((), ())),
                               precision=_HIGHEST)
    vn_n = jnp.sqrt(vh96[:, 0:NF] ** 2 + vh96[:, NF:2 * NF] ** 2 +
                    vh96[:, 2 * NF:3 * NF] ** 2 + 1e-8)
    so_n = jax.lax.dot_general(
        jnp.concatenate([V_dih, vn_n], axis=-1), wsnw_ref[...],
        (((1,), (0,)), ((), ())), precision=_HIGHEST) + wsnb_ref[...]
    W96 = _blockdiag3(wvn_ref[...], NF)
    vo96 = jax.lax.dot_general(vh96, W96, (((1,), (0,)), ((), ())),
                               precision=_HIGHEST)
    vs = _ln(so_n, gn_ref[...], bn_ref[...])
    v_out_ref[0, 0] = jnp.concatenate([vo96, vs], axis=-1)


def kernel(X, mask, Wh_n, Wv_n, Wsn_w, Wsn_b, Wh_e, Wv_e, Wse_w, Wse_b,
           g_n, b_n, g_e, b_e, chain_idx, batched_focuses):
    del mask
    Xr = X.reshape(B, T, N, 12)
    Xt = jnp.swapaxes(Xr, 2, 3)
    focT = batched_focuses.astype(jnp.float32).reshape(B, T, N, 1)
    ch = chain_idx.astype(jnp.float32).reshape(B, 1, L)
    q = np.arange(N * NP)
    const = jnp.asarray(np.concatenate(
        [(q[:, None] // NP == np.arange(N)[None, :]).astype(np.float32),
         (q[:, None] % NP).astype(np.float32),
         np.zeros((N * NP, 1), np.float32)], axis=1))
    ifr = np.arange(N)[:, None] * np.exp(
        np.arange(8)[None, :] * -(2.0 * np.log(10000.0) / NPE))
    trig = jnp.asarray(np.concatenate(
        [np.cos(ifr), np.sin(ifr)], axis=1).astype(np.float32))

    def row(w):
        return w.reshape(1, -1)

    full = lambda shape: pl.BlockSpec(shape, lambda b, t: (0,) * len(shape))
    in_specs = [
        pl.BlockSpec((1, 1, N, 12), lambda b, t: (b, t, 0, 0)),
        pl.BlockSpec((1, 1, 12, N), lambda b, t: (b, t, 0, 0)),
        pl.BlockSpec((1, 1, N, 1), lambda b, t: (b, t, 0, 0)),
        pl.BlockSpec((1, 1, L), lambda b, t: (b, 0, 0)),
        full((N * NP, NP)),
        full((N, 16)),
        full((3, NF)),
        full((NF, NF)),
        full((6 + NF, NF)),
        full((1, NF)),
        full((1, NF)),
        full((NF, EF)),
        full((NRBF * 2 + NF, EF)),
        full((1, EF)),
        full((1, NF)),
        full((1, NF)),
        full((1, EF)),
        full((1, EF)),
    ]
    out_specs = [
        pl.BlockSpec((1, 1, N, 4 * NF), lambda b, t: (b, t, 0, 0)),
        pl.BlockSpec((1, 1, N, N, 4 * EF), lambda b, t: (b, t, 0, 0, 0)),
        pl.BlockSpec((1, 1, N, N, 1), lambda b, t: (b, t, 0, 0, 0)),
        pl.BlockSpec((1, 1, N, N, 1), lambda b, t: (b, t, 0, 0, 0)),
    ]
    out_shapes = [
        jax.ShapeDtypeStruct((B, T, N, 4 * NF), jnp.float32),
        jax.ShapeDtypeStruct((B, T, N, N, 4 * EF), jnp.float32),
        jax.ShapeDtypeStruct((B, T, N, N, 1), jnp.float32),
        jax.ShapeDtypeStruct((B, T, N, N, 1), jnp.float32),
    ]
    V, E, eidx_f, aidx_f = pl.pallas_call(
        _tile_kernel,
        grid=(B, T),
        in_specs=in_specs,
        out_specs=out_specs,
        out_shape=out_shapes,
    )(Xr, Xt, focT, ch, const, trig, Wh_n, Wv_n, Wsn_w, row(Wsn_b), row(Wh_e.reshape(-1)),
      Wv_e, Wse_w, row(Wse_b), row(g_n), row(b_n), row(g_e), row(b_e))
    return (V, E,
            eidx_f.reshape(B, T, N, N).astype(jnp.int32),
            aidx_f.reshape(B, T, N, N).astype(jnp.int32))


# speedup vs baseline: 1.3166x; 1.0944x over previous
import numpy as np
import jax
import jax.numpy as jnp
from jax.experimental import pallas as pl

B, T, N, L = 4, 50, 30, 500
NPE = 16
NRBF = 16
NF = 32
EF = 32

_HIGHEST = jax.lax.Precision.HIGHEST


def _iota(shape, dim):
    return jax.lax.broadcasted_iota(jnp.int32, shape, dim).astype(jnp.float32)


def _nrm(x, eps=1e-12):
    n = jnp.sqrt(jnp.sum(x * x, axis=-1, keepdims=True))
    return x / jnp.maximum(n, eps)


def _cross(u, v):
    ux, uy, uz = u[..., 0:1], u[..., 1:2], u[..., 2:3]
    vx, vy, vz = v[..., 0:1], v[..., 1:2], v[..., 2:3]
    return jnp.concatenate(
        [uy * vz - uz * vy, uz * vx - ux * vz, ux * vy - uy * vx], axis=-1)


def _ln(x, g, b, eps=1e-5):
    mu = jnp.mean(x, axis=-1, keepdims=True)
    var = jnp.mean((x - mu) ** 2, axis=-1, keepdims=True)
    return g * (x - mu) / jnp.sqrt(var + eps) + b


def _dihed(u2, u1, u0):
    n2 = _nrm(_cross(u2, u1))
    n1 = _nrm(_cross(u1, u0))
    cosD = jnp.clip(jnp.sum(n2 * n1, axis=-1, keepdims=True),
                    -1.0 + 1e-7, 1.0 - 1e-7)
    sg_in = jnp.sum(u2 * n1, axis=-1, keepdims=True)
    sg = jnp.where(sg_in > 0.0, 1.0, jnp.where(sg_in < 0.0, -1.0, 0.0))
    return cosD, sg * jnp.sqrt(1.0 - cosD * cosD)


def _blockdiag3(w, k):
    z = jnp.zeros_like(w)
    rows = [jnp.concatenate([w if j == a else z for j in range(3)], axis=1)
            for a in range(3)]
    return jnp.concatenate(rows, axis=0)


NP = 32
NE = N * NP


def _dot(a, b, precision=_HIGHEST):
    return jax.lax.dot_general(a, b, (((1,), (0,)), ((), ())),
                               precision=precision)


def _tile_kernel(x_ref, xt_ref, focT_ref, ch_ref, const_ref, trig_ref,
                 whn_ref, wvn_ref, wsnw_ref, wsnb_ref,
                 whe_ref, wve_ref, wsew_ref, wseb_ref,
                 gn_ref, bn_ref, ge_ref, be_ref,
                 v_out_ref, e_out_ref, eidx_ref, aidx_ref):
    x = x_ref[0, 0]
    xt = xt_ref[0, 0]
    n_a = x[:, 0:3]
    ca = x[:, 3:6]
    c_a = x[:, 6:9]
    focT = focT_ref[0, 0]
    chain = ch_ref[0]
    E30 = const_ref[:, 0:N]
    r_flat = const_ref[:, N:N + 1]

    dx = ca[:, 0:1] - xt[3:4, :]
    dy = ca[:, 1:2] - xt[4:5, :]
    dz = ca[:, 2:3] - xt[5:6, :]
    D = jnp.sqrt(dx * dx + dy * dy + dz * dz + 1e-6)

    A = D[:, None, :]
    Bq = D[:, :, None]
    lt = (A < Bq).astype(jnp.float32)
    ktri = _iota((1, N, N), 2) < _iota((1, N, N), 1)
    eq = jnp.logical_and(A == Bq, ktri).astype(jnp.float32)
    rank = jnp.sum(lt + eq, axis=-1)

    rank_exp = _dot(E30, rank, precision=None)
    Pm = (rank_exp == r_flat).astype(jnp.float32)

    oh = (focT == _iota((N, L), 1)).astype(jnp.float32)
    nodechain = jnp.sum(oh * chain, axis=-1, keepdims=True)
    P0 = (rank == 0.0).astype(jnp.float32)
    first_chain = _dot(P0, nodechain)

    freq8 = jnp.exp(_iota((1, 8), 1) * (-2.0 * np.log(10000.0) / NPE))
    fa = focT * freq8
    cjsj = jnp.cos(jnp.concatenate([fa, fa - np.float32(np.pi / 2)], axis=-1))
    cisi = trig_ref[...]

    z1 = jnp.zeros((N, 1), jnp.float32)
    z16 = jnp.zeros((N, 16), jnp.float32)
    i_col = _iota((N, 1), 0)
    top = jnp.concatenate([ca, i_col, focT, nodechain, cjsj, z16], axis=-1)
    bot = jnp.concatenate([-ca, z1, z1, -first_chain, z16, cisi], axis=-1)
    rhs = jnp.concatenate([top, bot], axis=0)
    lhs = jnp.concatenate([Pm, E30], axis=-1)
    rhs_hi = rhs.astype(jnp.bfloat16).astype(jnp.float32)
    G = (_dot(lhs, rhs_hi, precision=None)
         + _dot(lhs, rhs - rhs_hi, precision=None))

    flo = jnp.floor(G[:, 3:5] + 0.5)
    eidx_col = flo[:, 0:1]
    aidx_col = flo[:, 1:2]
    same = (jnp.abs(G[:, 5:6]) < 0.5).astype(jnp.float32)
    same16 = _dot(same, jnp.ones((1, NRBF), jnp.float32), precision=None)

    eidx_ref[0, 0] = eidx_col.reshape(N, NP, 1)[:, 0:N, :]
    aidx_ref[0, 0] = aidx_col.reshape(N, NP, 1)[:, 0:N, :]

    dnb = G[:, 0:3]
    ssq = _dot(dnb * dnb, jnp.ones((3, 1), jnp.float32))
    sq2 = jnp.sqrt(jnp.concatenate([ssq + 1e-6, ssq], axis=-1))
    Dn = sq2[:, 0:1]
    inv = 1.0 / jnp.maximum(sq2[:, 1:2], 1e-12)
    Ed = dnb * inv
    dir_ssq = ssq * (inv * inv)

    sig = 20.0 / NRBF
    lane16 = _iota((1, 16), 1)
    rbf_arg = _dot(Dn, jnp.full((1, 16), 1.0 / sig, jnp.float32)) \
        - lane16 * ((20.0 / (NRBF - 1)) / sig)
    RBF = jnp.exp(-(rbf_arg * rbf_arg))
    cjE, sjE = G[:, 6:14], G[:, 14:22]
    ciE, siE = G[:, 22:30], G[:, 30:38]
    Epos = jnp.concatenate([cjE * ciE + sjE * siE,
                            sjE * ciE - cjE * siE], axis=-1) * same16

    whe = whe_ref[...]
    w2 = _dot(whe, wve_ref[...])
    zr = jnp.zeros_like(w2)
    wexp = jnp.concatenate([
        jnp.concatenate([w2, zr, zr, zr], axis=-1),
        jnp.concatenate([zr, w2, zr, zr], axis=-1),
        jnp.concatenate([zr, zr, w2, zr], axis=-1),
        jnp.concatenate([zr, zr, zr, whe * whe], axis=-1)], axis=0)
    G2 = _dot(jnp.concatenate([Ed, dir_ssq], axis=-1), wexp)
    vo_e = G2[:, 0:96]
    vn_e = jnp.sqrt(G2[:, 96:128] + 1e-8)

    cat_e = jnp.concatenate([RBF, Epos, vn_e], axis=-1)
    so_e = _dot(cat_e, wsew_ref[...]) + wseb_ref[...]
    es = _ln(so_e, ge_ref[...], be_ref[...])
    e_full = jnp.concatenate([vo_e, es], axis=-1)
    e_out_ref[0, 0] = e_full.reshape(N, NP, 4 * EF)[:, 0:N, :]

    b0 = _nrm(ca - n_a)
    b1 = _nrm(c_a - ca)
    b2v = _nrm(n_a[1:] - c_a[:-1])
    zrow = jnp.zeros((1, 3), jnp.float32)
    U2 = jnp.concatenate([b2v, zrow], axis=0)
    U0n = jnp.concatenate([b0[1:], zrow], axis=0)
    U2p = jnp.concatenate([zrow, b2v], axis=0)
    c1, s1 = _dihed(b0, b1, U2)
    c2, s2 = _dihed(b1, U2, U0n)
    c0, s0 = _dihed(U2p, b0, b1)
    row = _iota((N, 1), 0)
    lo, hi = row >= 1.0, row <= (N - 2.0)
    cos3 = jnp.concatenate([jnp.where(lo, c0, 1.0), jnp.where(hi, c1, 1.0),
                            jnp.where(hi, c2, 1.0)], axis=-1)
    sin3 = jnp.concatenate([jnp.where(lo, s0, 0.0), jnp.where(hi, s1, 0.0),
                            jnp.where(hi, s2, 0.0)], axis=-1)
    V_dih = jnp.concatenate([cos3, sin3], axis=-1)

    fw_core = _nrm(ca[1:] - ca[:-1])
    fw = jnp.concatenate([fw_core, zrow], axis=0)
    bw = jnp.concatenate([zrow, -fw_core], axis=0)

    cdir = _nrm(c_a - ca)
    ndir = _nrm(n_a - ca)
    bis = _nrm(cdir + ndir)
    perp = _nrm(_cross(cdir, ndir))
    vec = -bis * np.sqrt(1.0 / 3.0) - perp * np.sqrt(2.0 / 3.0)

    vparts = []
    for a in range(3):
        vparts += [vec[:, a:a + 1], fw[:, a:a + 1], bw[:, a:a + 1]]
    v_n = jnp.concatenate(vparts, axis=-1)

    W9 = _blockdiag3(whn_ref[...], 3)
    vh96 = jax.lax.dot_general(v_n, W9, (((1,), (0,)), ((), ())),
                               precision=_HIGHEST)
    vn_n = jnp.sqrt(vh96[:, 0:NF] ** 2 + vh96[:, NF:2 * NF] ** 2 +
                    vh96[:, 2 * NF:3 * NF] ** 2 + 1e-8)
    so_n = jax.lax.dot_general(
        jnp.concatenate([V_dih, vn_n], axis=-1), wsnw_ref[...],
        (((1,), (0,)), ((), ())), precision=_HIGHEST) + wsnb_ref[...]
    W96 = _blockdiag3(wvn_ref[...], NF)
    vo96 = jax.lax.dot_general(vh96, W96, (((1,), (0,)), ((), ())),
                               precision=_HIGHEST)
    vs = _ln(so_n, gn_ref[...], bn_ref[...])
    v_out_ref[0, 0] = jnp.concatenate([vo96, vs], axis=-1)


def kernel(X, mask, Wh_n, Wv_n, Wsn_w, Wsn_b, Wh_e, Wv_e, Wse_w, Wse_b,
           g_n, b_n, g_e, b_e, chain_idx, batched_focuses):
    del mask
    Xr = X.reshape(B, T, N, 12)
    Xt = jnp.swapaxes(Xr, 2, 3)
    focT = batched_focuses.astype(jnp.float32).reshape(B, T, N, 1)
    ch = chain_idx.astype(jnp.float32).reshape(B, 1, L)
    q = np.arange(N * NP)
    const = jnp.asarray(np.concatenate(
        [(q[:, None] // NP == np.arange(N)[None, :]).astype(np.float32),
         (q[:, None] % NP).astype(np.float32),
         np.zeros((N * NP, 1), np.float32)], axis=1))
    ifr = np.arange(N)[:, None] * np.exp(
        np.arange(8)[None, :] * -(2.0 * np.log(10000.0) / NPE))
    trig = jnp.asarray(np.concatenate(
        [np.cos(ifr), np.sin(ifr)], axis=1).astype(np.float32))

    def row(w):
        return w.reshape(1, -1)

    full = lambda shape: pl.BlockSpec(shape, lambda b, t: (0,) * len(shape))
    in_specs = [
        pl.BlockSpec((1, 1, N, 12), lambda b, t: (b, t, 0, 0)),
        pl.BlockSpec((1, 1, 12, N), lambda b, t: (b, t, 0, 0)),
        pl.BlockSpec((1, 1, N, 1), lambda b, t: (b, t, 0, 0)),
        pl.BlockSpec((1, 1, L), lambda b, t: (b, 0, 0)),
        full((N * NP, NP)),
        full((N, 16)),
        full((3, NF)),
        full((NF, NF)),
        full((6 + NF, NF)),
        full((1, NF)),
        full((1, NF)),
        full((NF, EF)),
        full((NRBF * 2 + NF, EF)),
        full((1, EF)),
        full((1, NF)),
        full((1, NF)),
        full((1, EF)),
        full((1, EF)),
    ]
    out_specs = [
        pl.BlockSpec((1, 1, N, 4 * NF), lambda b, t: (b, t, 0, 0)),
        pl.BlockSpec((1, 1, N, N, 4 * EF), lambda b, t: (b, t, 0, 0, 0)),
        pl.BlockSpec((1, 1, N, N, 1), lambda b, t: (b, t, 0, 0, 0)),
        pl.BlockSpec((1, 1, N, N, 1), lambda b, t: (b, t, 0, 0, 0)),
    ]
    out_shapes = [
        jax.ShapeDtypeStruct((B, T, N, 4 * NF), jnp.float32),
        jax.ShapeDtypeStruct((B, T, N, N, 4 * EF), jnp.float32),
        jax.ShapeDtypeStruct((B, T, N, N, 1), jnp.float32),
        jax.ShapeDtypeStruct((B, T, N, N, 1), jnp.float32),
    ]
    V, E, eidx_f, aidx_f = pl.pallas_call(
        _tile_kernel,
        grid=(B, T),
        in_specs=in_specs,
        out_specs=out_specs,
        out_shape=out_shapes,
    )(Xr, Xt, focT, ch, const, trig, Wh_n, Wv_n, Wsn_w, row(Wsn_b), row(Wh_e.reshape(-1)),
      Wv_e, Wse_w, row(Wse_b), row(g_n), row(b_n), row(g_e), row(b_e))
    return (V, E,
            eidx_f.reshape(B, T, N, N).astype(jnp.int32),
            aidx_f.reshape(B, T, N, N).astype(jnp.int32))
